# baseline (speedup 1.0000x reference)
import functools

import jax
import jax.numpy as jnp
from jax import lax
from jax.experimental import pallas as pl
from jax.experimental.pallas import tpu as pltpu
from jax.experimental.pallas import tpu_sc as plsc

_CHUNK = 64


def _segment_sum_sc(gidx2, dst2, hcat, zeros, acc_n):
    total_chunks, chunk = gidx2.shape
    aw = zeros.shape[1]
    info = plsc.get_sparse_core_info()
    nc, ns = info.num_cores, info.num_subcores
    nw = nc * ns
    cpw = total_chunks // nw
    rows_per_tile = acc_n // ns
    nlane = info.num_lanes
    mesh = plsc.VectorSubcoreMesh(core_axis_name="c", subcore_axis_name="s")

    nbuf = 2
    nstage = 5
    spc = cpw // nstage
    assert cpw % (nstage * nbuf) == 0 and spc % 8 == 0

    @functools.partial(
        pl.kernel,
        mesh=mesh,
        out_type=jax.ShapeDtypeStruct((nc, acc_n, aw), jnp.float32),
        scratch_types=[
            pltpu.VMEM((spc, chunk), jnp.int32),
            pltpu.VMEM((spc, chunk), jnp.int32),
            pltpu.VMEM((chunk, aw), jnp.float32),
            pltpu.VMEM_SHARED((acc_n, aw), jnp.float32),
        ]
        + [pltpu.VMEM((chunk, 2 * aw), jnp.float32) for _ in range(nbuf)]
        + [pltpu.SemaphoreType.DMA for _ in range(nbuf)],
    )
    def seg_kernel(gidx_hbm, dst_hbm, h_hbm, z_hbm, out_hbm,
                   gidx_v, dst_v, cbuf, acc, *bufs_and_sems):
        rows = bufs_and_sems[:nbuf]
        sems = bufs_and_sems[nbuf:]
        c = lax.axis_index("c")
        s = lax.axis_index("s")
        w = c * ns + s
        tile_rows = pl.ds(s * rows_per_tile, rows_per_tile)
        pltpu.sync_copy(z_hbm, acc.at[tile_rows])
        plsc.subcore_barrier()

        def do_chunk(j, b):
            pltpu.make_async_copy(h_hbm.at[gidx_v.at[0]], rows[b],
                                  sems[b]).wait()

            @plsc.parallel_loop(0, chunk, unroll=8)
            def compact(r):
                for k in range(aw // nlane):
                    cbuf[r, pl.ds(k * nlane, nlane)] = (
                        rows[b][r, pl.ds(k * nlane, nlane)])

            pltpu.sync_copy(cbuf, acc.at[dst_v.at[j]], add=True)

        for stage in range(nstage):
            base = w * cpw + stage * spc
            pltpu.sync_copy(gidx_hbm.at[pl.ds(base, spc)], gidx_v)
            pltpu.sync_copy(dst_hbm.at[pl.ds(base, spc)], dst_v)
            for b in range(nbuf):
                pltpu.async_copy(h_hbm.at[gidx_v.at[b]], rows[b], sems[b])

            def round_body(g, carry):
                for b in range(nbuf):
                    j = g * nbuf + b
                    do_chunk(j, b)
                    pltpu.async_copy(h_hbm.at[gidx_v.at[j + nbuf]], rows[b],
                                     sems[b])
                return carry

            lax.fori_loop(0, spc // nbuf - 1, round_body, 0)
            for b in range(nbuf):
                j = (spc // nbuf - 1) * nbuf + b
                do_chunk(j, b)
        plsc.subcore_barrier()
        pltpu.sync_copy(acc.at[tile_rows], out_hbm.at[c, tile_rows])

    return seg_kernel(gidx2, dst2, hcat, zeros)


def _mlp_body(h_ref, part_ref, eps_ref, w1_ref, b1_ref, g_ref, be_ref,
              w2_ref, b2_ref, out_ref):
    n = h_ref.shape[0]
    h = h_ref[...]
    agg = part_ref[0, :n, :]
    for i in range(1, part_ref.shape[0]):
        agg = agg + part_ref[i, :n, :]
    z = h + eps_ref[...] * h + agg
    p = jnp.dot(z, w1_ref[...], preferred_element_type=jnp.float32) + b1_ref[...]
    m = jnp.sum(p, axis=0, keepdims=True) * (1.0 / n)
    pc = p - m
    v = jnp.sum(pc * pc, axis=0, keepdims=True) * (1.0 / n)
    q = pc * lax.rsqrt(v + 1e-5) * g_ref[...] + be_ref[...]
    q = jnp.maximum(q, 0.0)
    out_ref[...] = (
        jnp.dot(q, w2_ref[...], preferred_element_type=jnp.float32) + b2_ref[...]
    )


def _mlp_tc(h, partials, eps, w1, b1, g, be, w2, b2):
    n, d = h.shape
    return pl.pallas_call(
        _mlp_body,
        out_shape=jax.ShapeDtypeStruct((n, d), jnp.float32),
    )(h, partials, eps.reshape(1, 1), w1, b1.reshape(1, -1), g.reshape(1, -1),
      be.reshape(1, -1), w2, b2.reshape(1, -1))


def kernel(x, edge_index, W1a, b1a, g1a, be1a, W2a, b2a, eps_a,
           W1b, b1b, g1b, be1b, W2b, b2b, eps_b):
    n, d = x.shape
    e = edge_index.shape[1]
    info = plsc.get_sparse_core_info()
    nc, ns = info.num_cores, info.num_subcores
    nw = nc * ns
    cpw = -(-e // (nw * _CHUNK * 8)) * 8
    epad = nw * cpw * _CHUNK
    acc_n = -(-(n + 1) // (ns * 8)) * (ns * 8)
    src = edge_index[0]
    dst = edge_index[1]
    pad = epad - e
    if pad:
        src = jnp.concatenate([src, jnp.arange(pad, dtype=jnp.int32) % n])
        dst = jnp.concatenate(
            [dst, n + jnp.arange(pad, dtype=jnp.int32) % (acc_n - n)])
    gidx2 = (jnp.right_shift(src, 1) + jnp.bitwise_and(src, 1) * (n // 2)
             ).reshape(epad // _CHUNK, _CHUNK)
    dst2 = dst.reshape(epad // _CHUNK, _CHUNK)
    zeros = jnp.zeros((acc_n // ns, d), jnp.float32)

    def seg(h):
        hcat = jnp.concatenate(
            [h, jnp.concatenate([h[1:], h[:1]])]).reshape(n, 2 * d)
        return _segment_sum_sc(gidx2, dst2, hcat, zeros, acc_n)

    part_a = seg(x)
    h1 = _mlp_tc(x, part_a, eps_a, W1a, b1a, g1a, be1a, W2a, b2a)
    part_b = seg(h1)
    return _mlp_tc(h1, part_b, eps_b, W1b, b1b, g1b, be1b, W2b, b2b)


# speedup vs baseline: 1.0671x; 1.0671x over previous
import functools

import jax
import jax.numpy as jnp
from jax import lax
from jax.experimental import pallas as pl
from jax.experimental.pallas import tpu as pltpu
from jax.experimental.pallas import tpu_sc as plsc

_CHUNK = 64


def _segment_sum_sc(gidx2, dst2, hcat, zeros, acc_n):
    total_chunks, chunk = gidx2.shape
    aw = zeros.shape[1]
    info = plsc.get_sparse_core_info()
    nc, ns = info.num_cores, info.num_subcores
    nw = nc * ns
    cpw = total_chunks // nw
    rows_per_tile = acc_n // ns
    nlane = info.num_lanes
    mesh = plsc.VectorSubcoreMesh(core_axis_name="c", subcore_axis_name="s")

    nbuf = 2
    nstage = 5
    spc = cpw // nstage
    assert cpw % (nstage * nbuf) == 0 and spc % 8 == 0

    @functools.partial(
        pl.kernel,
        mesh=mesh,
        out_type=jax.ShapeDtypeStruct((nc, acc_n, aw), jnp.float32),
        scratch_types=[
            pltpu.VMEM((spc, chunk), jnp.int32),
            pltpu.VMEM((spc, chunk), jnp.int32),
            pltpu.VMEM((chunk, aw), jnp.float32),
            pltpu.VMEM_SHARED((acc_n, aw), jnp.float32),
        ]
        + [pltpu.VMEM((chunk, 2 * aw), jnp.float32) for _ in range(nbuf)]
        + [pltpu.SemaphoreType.DMA for _ in range(nbuf)],
    )
    def seg_kernel(gidx_hbm, dst_hbm, h_hbm, z_hbm, out_hbm,
                   gidx_v, dst_v, cbuf, acc, *bufs_and_sems):
        rows = bufs_and_sems[:nbuf]
        sems = bufs_and_sems[nbuf:]
        c = lax.axis_index("c")
        s = lax.axis_index("s")
        w = c * ns + s
        tile_rows = pl.ds(s * rows_per_tile, rows_per_tile)
        pltpu.sync_copy(z_hbm, acc.at[tile_rows])
        plsc.subcore_barrier()

        def do_chunk(j, b, refill_j=None):
            pltpu.make_async_copy(h_hbm.at[gidx_v.at[0]], rows[b],
                                  sems[b]).wait()

            @plsc.parallel_loop(0, chunk, unroll=4)
            def compact(r):
                for k in range(aw // nlane):
                    cbuf[r, pl.ds(k * nlane, nlane)] = (
                        rows[b][r, pl.ds(k * nlane, nlane)])

            if refill_j is not None:
                pltpu.async_copy(h_hbm.at[gidx_v.at[refill_j]], rows[b],
                                 sems[b])
            pltpu.sync_copy(cbuf, acc.at[dst_v.at[j]], add=True)

        for stage in range(nstage):
            base = w * cpw + stage * spc
            pltpu.sync_copy(gidx_hbm.at[pl.ds(base, spc)], gidx_v)
            pltpu.sync_copy(dst_hbm.at[pl.ds(base, spc)], dst_v)
            for b in range(nbuf):
                pltpu.async_copy(h_hbm.at[gidx_v.at[b]], rows[b], sems[b])

            def round_body(g, carry):
                for b in range(nbuf):
                    j = g * nbuf + b
                    do_chunk(j, b, refill_j=j + nbuf)
                return carry

            lax.fori_loop(0, spc // nbuf - 1, round_body, 0)
            for b in range(nbuf):
                j = (spc // nbuf - 1) * nbuf + b
                do_chunk(j, b)
        plsc.subcore_barrier()
        pltpu.sync_copy(acc.at[tile_rows], out_hbm.at[c, tile_rows])

    return seg_kernel(gidx2, dst2, hcat, zeros)


def _mlp_body(h_ref, part_ref, eps_ref, w1_ref, b1_ref, g_ref, be_ref,
              w2_ref, b2_ref, out_ref):
    n = h_ref.shape[0]
    h = h_ref[...]
    agg = part_ref[0, :n, :]
    for i in range(1, part_ref.shape[0]):
        agg = agg + part_ref[i, :n, :]
    z = h + eps_ref[...] * h + agg
    p = jnp.dot(z, w1_ref[...], preferred_element_type=jnp.float32) + b1_ref[...]
    m = jnp.sum(p, axis=0, keepdims=True) * (1.0 / n)
    pc = p - m
    v = jnp.sum(pc * pc, axis=0, keepdims=True) * (1.0 / n)
    q = pc * lax.rsqrt(v + 1e-5) * g_ref[...] + be_ref[...]
    q = jnp.maximum(q, 0.0)
    out_ref[...] = (
        jnp.dot(q, w2_ref[...], preferred_element_type=jnp.float32) + b2_ref[...]
    )


def _mlp_tc(h, partials, eps, w1, b1, g, be, w2, b2):
    n, d = h.shape
    return pl.pallas_call(
        _mlp_body,
        out_shape=jax.ShapeDtypeStruct((n, d), jnp.float32),
    )(h, partials, eps.reshape(1, 1), w1, b1.reshape(1, -1), g.reshape(1, -1),
      be.reshape(1, -1), w2, b2.reshape(1, -1))


def kernel(x, edge_index, W1a, b1a, g1a, be1a, W2a, b2a, eps_a,
           W1b, b1b, g1b, be1b, W2b, b2b, eps_b):
    n, d = x.shape
    e = edge_index.shape[1]
    info = plsc.get_sparse_core_info()
    nc, ns = info.num_cores, info.num_subcores
    nw = nc * ns
    cpw = -(-e // (nw * _CHUNK * 8)) * 8
    epad = nw * cpw * _CHUNK
    acc_n = -(-(n + 1) // (ns * 8)) * (ns * 8)
    src = edge_index[0]
    dst = edge_index[1]
    pad = epad - e
    if pad:
        src = jnp.concatenate([src, jnp.arange(pad, dtype=jnp.int32) % n])
        dst = jnp.concatenate(
            [dst, n + jnp.arange(pad, dtype=jnp.int32) % (acc_n - n)])
    gidx2 = (jnp.right_shift(src, 1) + jnp.bitwise_and(src, 1) * (n // 2)
             ).reshape(epad // _CHUNK, _CHUNK)
    dst2 = dst.reshape(epad // _CHUNK, _CHUNK)
    zeros = jnp.zeros((acc_n // ns, d), jnp.float32)

    def seg(h):
        hcat = jnp.concatenate(
            [h, jnp.concatenate([h[1:], h[:1]])]).reshape(n, 2 * d)
        return _segment_sum_sc(gidx2, dst2, hcat, zeros, acc_n)

    part_a = seg(x)
    h1 = _mlp_tc(x, part_a, eps_a, W1a, b1a, g1a, be1a, W2a, b2a)
    part_b = seg(h1)
    return _mlp_tc(h1, part_b, eps_b, W1b, b1b, g1b, be1b, W2b, b2b)
